# Initial kernel scaffold; baseline (speedup 1.0000x reference)
import jax
import jax.numpy as jnp
from jax.experimental import pallas as pl


def kernel(target, grid, sht_weights):
    raise NotImplementedError("write your pallas kernel here")


# speedup vs baseline: 37.3002x; 37.3002x over previous
import math
import numpy as np
import jax
import jax.numpy as jnp
from jax.experimental import pallas as pl

_NLAT = 128
_NLON = 256
_LMAX = 50
_MMAX = 50
_NREF = 2048
_NQ = _NLAT * _NLON
_QT = 512
_NQT = _NQ // _QT


def _knn_interp_body(qt_ref, qp_ref, rt_ref, rp_ref, rr_ref, out_ref):
    qt = qt_ref[0]
    qp = qp_ref[0]
    rt = rt_ref[0]
    rp = rp_ref[0]
    rr = rr_ref[0]

    dth = rt - qt
    dph = rp - qp
    d2 = dth * dth + dph * dph

    iot = jax.lax.broadcasted_iota(jnp.int32, (_NREF, _QT), 0)
    bigf = jnp.float32(3.0e38)
    bigi = jnp.int32(2 ** 30)

    dsum = jnp.zeros((1, _QT), jnp.float32)
    acc = jnp.zeros((1, _QT), jnp.float32)
    for k in range(3):
        m = jnp.min(d2, axis=0, keepdims=True)
        cand = jnp.where(d2 == m, iot, bigi)
        idx = jnp.min(cand, axis=0, keepdims=True)
        oh = iot == idx
        rk = jnp.sum(jnp.where(oh, rr, 0.0), axis=0, keepdims=True)
        dk = jnp.sqrt(m)
        dsum = dsum + dk
        acc = acc + dk * rk
        if k < 2:
            d2 = jnp.where(oh, bigf, d2)

    out_ref[...] = (acc / dsum).reshape(1, 1, 1, _QT)


def _sht_body(f_ref, c_ref, w_ref, o_ref):
    f = f_ref[0]
    x = jnp.dot(f, c_ref[...], preferred_element_type=jnp.float32,
                precision=jax.lax.Precision.HIGHEST)
    t = w_ref[...] * x[:, None, :]
    o_ref[...] = jnp.sum(t, axis=0).reshape(1, _LMAX, _MMAX)


def _cos_matrix():
    n = np.arange(_NLON)[:, None].astype(np.float64)
    m = np.arange(_MMAX)[None, :].astype(np.float64)
    c = (2.0 * np.pi / _NLON) * np.cos(2.0 * np.pi * m * n / _NLON)
    return jnp.asarray(c.astype(np.float32))


def kernel(target, grid, sht_weights):
    x, y, z = target[..., 0], target[..., 1], target[..., 2]
    r = jnp.sqrt(x * x + y * y + z * z)
    theta = jnp.arccos(x / r)
    nzy = jnp.sqrt(z * z + y * y)
    a = jnp.arccos(y / nzy)
    phi = a + (2.0 * math.pi - 2.0 * a) * (z < 0).astype(jnp.float32)
    phi = phi - math.pi

    qt = grid[0, :, 0].reshape(_NQT, 1, _QT)
    qp = grid[0, :, 1].reshape(_NQT, 1, _QT)
    rt3 = theta.reshape(2, _NREF, 1)
    rp3 = phi.reshape(2, _NREF, 1)
    rr3 = r.reshape(2, _NREF, 1)

    f = pl.pallas_call(
        _knn_interp_body,
        grid=(2, _NQT),
        in_specs=[
            pl.BlockSpec((1, 1, _QT), lambda b, t: (t, 0, 0)),
            pl.BlockSpec((1, 1, _QT), lambda b, t: (t, 0, 0)),
            pl.BlockSpec((1, _NREF, 1), lambda b, t: (b, 0, 0)),
            pl.BlockSpec((1, _NREF, 1), lambda b, t: (b, 0, 0)),
            pl.BlockSpec((1, _NREF, 1), lambda b, t: (b, 0, 0)),
        ],
        out_specs=pl.BlockSpec((1, 1, 1, _QT), lambda b, t: (b, t, 0, 0)),
        out_shape=jax.ShapeDtypeStruct((2, _NQT, 1, _QT), jnp.float32),
    )(qt, qp, rt3, rp3, rr3)

    fgrid = f.reshape(2, _NLAT, _NLON)
    cmat = _cos_matrix()
    w4 = jnp.transpose(sht_weights, (2, 1, 0))

    out = pl.pallas_call(
        _sht_body,
        grid=(2,),
        in_specs=[
            pl.BlockSpec((1, _NLAT, _NLON), lambda b: (b, 0, 0)),
            pl.BlockSpec((_NLON, _MMAX), lambda b: (0, 0)),
            pl.BlockSpec((_NLAT, _LMAX, _MMAX), lambda b: (0, 0, 0)),
        ],
        out_specs=pl.BlockSpec((1, _LMAX, _MMAX), lambda b: (b, 0, 0)),
        out_shape=jax.ShapeDtypeStruct((2, _LMAX, _MMAX), jnp.float32),
    )(fgrid, cmat, w4)
    return out
